# Initial kernel scaffold; baseline (speedup 1.0000x reference)
import jax
import jax.numpy as jnp
from jax.experimental import pallas as pl


def kernel(x, pos, spectral_filter, bn_gamma, bn_beta):
    raise NotImplementedError("write your pallas kernel here")


# speedup vs baseline: 14.5941x; 14.5941x over previous
import functools

import jax
import jax.numpy as jnp
from jax import lax
from jax.experimental import pallas as pl
from jax.experimental.pallas import tpu as pltpu
from jax.experimental.pallas import tpu_sc as plsc

_B, _C, _N, _K, _D = 4, 128, 4096, 16, 128
_R = 256
_NB = 512
_G = 128
_NC, _NS = 2, 16
_NW = _NC * _NS
_SB = 2048


def _knn_body(pos_all_ref, pos_blk_ref, idx_ref):
    p_all = pos_all_ref[0]
    p_blk = pos_blk_ref[0]
    inner = lax.dot_general(
        p_blk, p_all, (((0,), (0,)), ((), ())),
        preferred_element_type=jnp.float32)
    xx_all = jnp.sum(p_all * p_all, axis=0, keepdims=True)
    xx_blk = jnp.sum(p_blk * p_blk, axis=0)[:, None]
    nd = 2.0 * inner - xx_blk - xx_all
    iota = lax.broadcasted_iota(jnp.int32, (_R, _N), 1)
    for t in range(_K):
        m = jnp.max(nd, axis=1, keepdims=True)
        cand = jnp.where(nd == m, iota, _N)
        j = jnp.min(cand, axis=1, keepdims=True)
        idx_ref[0, t, :] = j[:, 0]
        nd = jnp.where(iota == j, -jnp.inf, nd)


def _knn_call(pos_pad):
    return pl.pallas_call(
        _knn_body,
        grid=(_B, _N // _R),
        in_specs=[
            pl.BlockSpec((1, 8, _N), lambda b, i: (b, 0, 0)),
            pl.BlockSpec((1, 8, _R), lambda b, i: (b, 0, i)),
        ],
        out_specs=pl.BlockSpec((1, _K, _R), lambda b, i: (b, 0, i)),
        out_shape=jax.ShapeDtypeStruct((_B, _K, _N), jnp.int32),
    )(pos_pad, pos_pad)


def _proj_body(x_ref, w_ref, y_ref, t0_ref):
    xb = x_ref[0]
    w = w_ref[...]
    for k in range(_K):
        y_ref[0, k] = lax.dot_general(
            xb, w[k], (((0,), (0,)), ((), ())),
            preferred_element_type=jnp.float32)
    wsum = jnp.sum(w, axis=0)
    t0_ref[0] = -lax.dot_general(
        xb, wsum, (((0,), (0,)), ((), ())),
        preferred_element_type=jnp.float32)


def _proj_call(x, w):
    return pl.pallas_call(
        _proj_body,
        grid=(_B, _N // _NB),
        in_specs=[
            pl.BlockSpec((1, _C, _NB), lambda b, i: (b, 0, i)),
            pl.BlockSpec((_K, _C, _D), lambda b, i: (0, 0, 0)),
        ],
        out_specs=[
            pl.BlockSpec((1, _K, _NB, _D), lambda b, i: (b, 0, i, 0)),
            pl.BlockSpec((1, _NB, _D), lambda b, i: (b, i, 0)),
        ],
        out_shape=[
            jax.ShapeDtypeStruct((_B, _K, _N, _D), jnp.float32),
            jax.ShapeDtypeStruct((_B, _N, _D), jnp.float32),
        ],
    )(x, w)


def _sc_gather_call(y, idx, t0):
    rows_per_w = _N // (_NW // _B)
    chunks = rows_per_w // _G
    wpb = _NW // _B

    mesh = plsc.VectorSubcoreMesh(core_axis_name="c", subcore_axis_name="s",
                                  num_cores=_NC, num_subcores=_NS)

    @functools.partial(
        pl.kernel,
        mesh=mesh,
        out_type=jax.ShapeDtypeStruct((_B, _N, _D), jnp.float32),
        scratch_types=[
            pltpu.VMEM((_K, _G), jnp.int32),
            pltpu.VMEM((_G, _D), jnp.float32),
            pltpu.SemaphoreType.DMA,
        ],
    )
    def sc_kernel(y_hbm, idx_hbm, t0_hbm, out_hbm, idxv, accv, sem):
        cid = lax.axis_index("c")
        sid = lax.axis_index("s")
        w = sid * _NC + cid
        b = w // wpb
        nbase = (w % wpb) * rows_per_w

        def chunk(j, carry):
            n0 = nbase + j * _G
            pltpu.sync_copy(idx_hbm.at[b, :, pl.ds(n0, _G)], idxv)
            pltpu.sync_copy(t0_hbm.at[b, pl.ds(n0, _G)], accv)
            descs = [
                pltpu.async_copy(y_hbm.at[b, k].at[idxv.at[k]], accv, sem,
                                 add=True)
                for k in range(_K)
            ]
            for d in descs:
                d.wait()
            pltpu.sync_copy(accv, out_hbm.at[b, pl.ds(n0, _G)])
            return carry

        lax.fori_loop(0, chunks, chunk, 0)

    return sc_kernel(y, idx, t0)


def _stats_body(o_ref, s_ref):
    @pl.when(pl.program_id(0) == 0)
    def _():
        s_ref[...] = jnp.zeros_like(s_ref)

    blk = o_ref[...]
    s_ref[0:1, :] += jnp.sum(blk, axis=0, keepdims=True)
    s_ref[1:2, :] += jnp.sum(blk * blk, axis=0, keepdims=True)


def _stats_call(o2d):
    return pl.pallas_call(
        _stats_body,
        grid=(_B * _N // _SB,),
        in_specs=[pl.BlockSpec((_SB, _D), lambda i: (i, 0))],
        out_specs=pl.BlockSpec((8, _D), lambda i: (0, 0)),
        out_shape=jax.ShapeDtypeStruct((8, _D), jnp.float32),
    )(o2d)


def _norm_body(o_ref, s_ref, gb_ref, y_ref):
    inv_n = jnp.float32(1.0 / (_B * _N))
    mean = s_ref[0:1, :] * inv_n
    var = s_ref[1:2, :] * inv_n - mean * mean
    scale = gb_ref[0:1, :] * lax.rsqrt(var + 1e-5)
    bias = gb_ref[1:2, :] - scale * mean
    y_ref[...] = o_ref[...] * scale + bias


def _norm_call(o2d, sums, gb):
    return pl.pallas_call(
        _norm_body,
        grid=(_B * _N // _SB,),
        in_specs=[
            pl.BlockSpec((_SB, _D), lambda i: (i, 0)),
            pl.BlockSpec((8, _D), lambda i: (0, 0)),
            pl.BlockSpec((8, _D), lambda i: (0, 0)),
        ],
        out_specs=pl.BlockSpec((_SB, _D), lambda i: (i, 0)),
        out_shape=jax.ShapeDtypeStruct((_B * _N, _D), jnp.float32),
    )(o2d, sums, gb)


def kernel(x, pos, spectral_filter, bn_gamma, bn_beta):
    pos_pad = jnp.concatenate(
        [pos, jnp.zeros((_B, 8 - pos.shape[1], _N), jnp.float32)], axis=1)
    idx = _knn_call(pos_pad)
    y, t0neg = _proj_call(x, spectral_filter)
    out = _sc_gather_call(y, idx, t0neg)
    o2d = out.reshape(_B * _N, _D)
    sums = _stats_call(o2d)
    gb = jnp.zeros((8, _D), jnp.float32)
    gb = gb.at[0].set(bn_gamma).at[1].set(bn_beta)
    normed = _norm_call(o2d, sums, gb)
    return jnp.transpose(normed.reshape(_B, _N, _D), (0, 2, 1))


# speedup vs baseline: 18.2526x; 1.2507x over previous
import functools

import jax
import jax.numpy as jnp
from jax import lax
from jax.experimental import pallas as pl
from jax.experimental.pallas import tpu as pltpu
from jax.experimental.pallas import tpu_sc as plsc

_B, _C, _N, _K, _D = 4, 128, 4096, 16, 128
_R = 256
_NB = 512
_G = 128
_NC, _NS = 2, 16
_NW = _NC * _NS
_SB = 2048


_CH = 4
_RC = _R // _CH


def _knn_body(pos_all_ref, pos_blk_ref, idx_ref):
    p_all = pos_all_ref[0]
    p_blk = pos_blk_ref[0]
    inner = lax.dot_general(
        p_blk, p_all, (((0,), (0,)), ((), ())),
        preferred_element_type=jnp.float32)
    xx_all = jnp.sum(p_all * p_all, axis=0, keepdims=True)
    xx_blk = jnp.sum(p_blk * p_blk, axis=0)[:, None]
    nd_full = 2.0 * inner - xx_blk - xx_all
    fiota = lax.broadcasted_iota(jnp.int32, (_RC, _N), 1).astype(jnp.float32)
    big = jnp.float32(_N)
    neg = jnp.float32(-jnp.inf)
    nds = [nd_full[c * _RC:(c + 1) * _RC] for c in range(_CH)]
    for t in range(_K):
        for c in range(_CH):
            nd = nds[c]
            m = jnp.max(nd, axis=1, keepdims=True)
            cand = jnp.where(nd == m, fiota, big)
            j = jnp.min(cand, axis=1, keepdims=True)
            idx_ref[0, c * _RC:(c + 1) * _RC, t:t + 1] = j.astype(jnp.int32)
            nds[c] = jnp.where(cand == j, neg, nd)


def _knn_call(pos_pad):
    return pl.pallas_call(
        _knn_body,
        grid=(_B, _N // _R),
        in_specs=[
            pl.BlockSpec((1, 8, _N), lambda b, i: (b, 0, 0)),
            pl.BlockSpec((1, 8, _R), lambda b, i: (b, 0, i)),
        ],
        out_specs=pl.BlockSpec((1, _R, _K), lambda b, i: (b, i, 0)),
        out_shape=jax.ShapeDtypeStruct((_B, _N, _K), jnp.int32),
    )(pos_pad, pos_pad)


def _proj_body(x_ref, w_ref, y_ref, t0_ref):
    xb = x_ref[0]
    w = w_ref[...]
    for k in range(_K):
        y_ref[0, k] = lax.dot_general(
            xb, w[k], (((0,), (0,)), ((), ())),
            preferred_element_type=jnp.float32)
    wsum = jnp.sum(w, axis=0)
    t0_ref[0] = -lax.dot_general(
        xb, wsum, (((0,), (0,)), ((), ())),
        preferred_element_type=jnp.float32)


def _proj_call(x, w):
    return pl.pallas_call(
        _proj_body,
        grid=(_B, _N // _NB),
        in_specs=[
            pl.BlockSpec((1, _C, _NB), lambda b, i: (b, 0, i)),
            pl.BlockSpec((_K, _C, _D), lambda b, i: (0, 0, 0)),
        ],
        out_specs=[
            pl.BlockSpec((1, _K, _NB, _D), lambda b, i: (b, 0, i, 0)),
            pl.BlockSpec((1, _NB, _D), lambda b, i: (b, i, 0)),
        ],
        out_shape=[
            jax.ShapeDtypeStruct((_B, _K, _N, _D), jnp.float32),
            jax.ShapeDtypeStruct((_B, _N, _D), jnp.float32),
        ],
    )(x, w)


def _sc_gather_call(y, idx, t0):
    rows_per_w = _N // (_NW // _B)
    chunks = rows_per_w // _G
    wpb = _NW // _B

    mesh = plsc.VectorSubcoreMesh(core_axis_name="c", subcore_axis_name="s",
                                  num_cores=_NC, num_subcores=_NS)

    @functools.partial(
        pl.kernel,
        mesh=mesh,
        out_type=jax.ShapeDtypeStruct((_B, _N, _D), jnp.float32),
        scratch_types=[
            pltpu.VMEM((_K, _G), jnp.int32),
            pltpu.VMEM((_G, _D), jnp.float32),
            pltpu.SemaphoreType.DMA,
        ],
    )
    def sc_kernel(y_hbm, idx_hbm, t0_hbm, out_hbm, idxv, accv, sem):
        cid = lax.axis_index("c")
        sid = lax.axis_index("s")
        w = sid * _NC + cid
        b = w // wpb
        nbase = (w % wpb) * rows_per_w

        def chunk(j, carry):
            n0 = nbase + j * _G
            pltpu.sync_copy(idx_hbm.at[b, :, pl.ds(n0, _G)], idxv)
            pltpu.sync_copy(t0_hbm.at[b, pl.ds(n0, _G)], accv)
            descs = [
                pltpu.async_copy(y_hbm.at[b, k].at[idxv.at[k]], accv, sem,
                                 add=True)
                for k in range(_K)
            ]
            for d in descs:
                d.wait()
            pltpu.sync_copy(accv, out_hbm.at[b, pl.ds(n0, _G)])
            return carry

        lax.fori_loop(0, chunks, chunk, 0)

    return sc_kernel(y, idx, t0)


def _stats_body(o_ref, s_ref):
    @pl.when(pl.program_id(0) == 0)
    def _():
        s_ref[...] = jnp.zeros_like(s_ref)

    blk = o_ref[...]
    s_ref[0:1, :] += jnp.sum(blk, axis=0, keepdims=True)
    s_ref[1:2, :] += jnp.sum(blk * blk, axis=0, keepdims=True)


def _stats_call(o2d):
    return pl.pallas_call(
        _stats_body,
        grid=(_B * _N // _SB,),
        in_specs=[pl.BlockSpec((_SB, _D), lambda i: (i, 0))],
        out_specs=pl.BlockSpec((8, _D), lambda i: (0, 0)),
        out_shape=jax.ShapeDtypeStruct((8, _D), jnp.float32),
    )(o2d)


def _norm_body(o_ref, s_ref, gb_ref, y_ref):
    inv_n = jnp.float32(1.0 / (_B * _N))
    mean = s_ref[0:1, :] * inv_n
    var = s_ref[1:2, :] * inv_n - mean * mean
    scale = gb_ref[0:1, :] * lax.rsqrt(var + 1e-5)
    bias = gb_ref[1:2, :] - scale * mean
    y_ref[...] = o_ref[...] * scale + bias


def _norm_call(o2d, sums, gb):
    return pl.pallas_call(
        _norm_body,
        grid=(_B * _N // _SB,),
        in_specs=[
            pl.BlockSpec((_SB, _D), lambda i: (i, 0)),
            pl.BlockSpec((8, _D), lambda i: (0, 0)),
            pl.BlockSpec((8, _D), lambda i: (0, 0)),
        ],
        out_specs=pl.BlockSpec((_SB, _D), lambda i: (i, 0)),
        out_shape=jax.ShapeDtypeStruct((_B * _N, _D), jnp.float32),
    )(o2d, sums, gb)


def kernel(x, pos, spectral_filter, bn_gamma, bn_beta):
    pos_pad = jnp.concatenate(
        [pos, jnp.zeros((_B, 8 - pos.shape[1], _N), jnp.float32)], axis=1)
    idx = jnp.transpose(_knn_call(pos_pad), (0, 2, 1))
    y, t0neg = _proj_call(x, spectral_filter)
    out = _sc_gather_call(y, idx, t0neg)
    o2d = out.reshape(_B * _N, _D)
    sums = _stats_call(o2d)
    gb = jnp.zeros((8, _D), jnp.float32)
    gb = gb.at[0].set(bn_gamma).at[1].set(bn_beta)
    normed = _norm_call(o2d, sums, gb)
    return jnp.transpose(normed.reshape(_B, _N, _D), (0, 2, 1))


# speedup vs baseline: 18.4120x; 1.0087x over previous
import functools

import jax
import jax.numpy as jnp
from jax import lax
from jax.experimental import pallas as pl
from jax.experimental.pallas import tpu as pltpu
from jax.experimental.pallas import tpu_sc as plsc

_B, _C, _N, _K, _D = 4, 128, 4096, 16, 128
_R = 256
_NB = 512
_G = 128
_NC, _NS = 2, 16
_NW = _NC * _NS
_SB = 2048


_CH = 4
_RC = _R // _CH


def _knn_body(pos_all_ref, pos_blk_ref, idx_ref):
    p_all = pos_all_ref[0]
    p_blk = pos_blk_ref[0]
    inner = lax.dot_general(
        p_blk, p_all, (((0,), (0,)), ((), ())),
        preferred_element_type=jnp.float32)
    xx_all = jnp.sum(p_all * p_all, axis=0, keepdims=True)
    xx_blk = jnp.sum(p_blk * p_blk, axis=0)[:, None]
    nd_full = 2.0 * inner - xx_blk - xx_all
    fiota = lax.broadcasted_iota(jnp.int32, (_RC, _N), 1).astype(jnp.float32)
    big = jnp.float32(_N)
    neg = jnp.float32(-jnp.inf)
    nds = [nd_full[c * _RC:(c + 1) * _RC] for c in range(_CH)]
    for t in range(_K):
        for c in range(_CH):
            nd = nds[c]
            m = jnp.max(nd, axis=1, keepdims=True)
            cand = jnp.where(nd == m, fiota, big)
            j = jnp.min(cand, axis=1, keepdims=True)
            idx_ref[0, c * _RC:(c + 1) * _RC, t:t + 1] = j.astype(jnp.int32)
            nds[c] = jnp.where(cand == j, neg, nd)


def _knn_call(pos_pad):
    return pl.pallas_call(
        _knn_body,
        grid=(_B, _N // _R),
        in_specs=[
            pl.BlockSpec((1, 8, _N), lambda b, i: (b, 0, 0)),
            pl.BlockSpec((1, 8, _R), lambda b, i: (b, 0, i)),
        ],
        out_specs=pl.BlockSpec((1, _R, _K), lambda b, i: (b, i, 0)),
        out_shape=jax.ShapeDtypeStruct((_B, _N, _K), jnp.int32),
    )(pos_pad, pos_pad)


def _proj_body(x_ref, w_ref, y_ref, t0_ref):
    xb = x_ref[0]
    w = w_ref[...]
    for k in range(_K):
        y_ref[0, k] = lax.dot_general(
            xb, w[k], (((0,), (0,)), ((), ())),
            preferred_element_type=jnp.float32)
    wsum = jnp.sum(w, axis=0)
    t0_ref[0] = -lax.dot_general(
        xb, wsum, (((0,), (0,)), ((), ())),
        preferred_element_type=jnp.float32)


def _proj_call(x, w):
    return pl.pallas_call(
        _proj_body,
        grid=(_B, _N // _NB),
        in_specs=[
            pl.BlockSpec((1, _C, _NB), lambda b, i: (b, 0, i)),
            pl.BlockSpec((_K, _C, _D), lambda b, i: (0, 0, 0)),
        ],
        out_specs=[
            pl.BlockSpec((1, _K, _NB, _D), lambda b, i: (b, 0, i, 0)),
            pl.BlockSpec((1, _NB, _D), lambda b, i: (b, i, 0)),
        ],
        out_shape=[
            jax.ShapeDtypeStruct((_B, _K, _N, _D), jnp.float32),
            jax.ShapeDtypeStruct((_B, _N, _D), jnp.float32),
        ],
    )(x, w)


def _sc_gather_call(y, idx, t0):
    rows_per_w = _N // (_NW // _B)
    chunks = rows_per_w // _G
    wpb = _NW // _B

    mesh = plsc.VectorSubcoreMesh(core_axis_name="c", subcore_axis_name="s",
                                  num_cores=_NC, num_subcores=_NS)

    @functools.partial(
        pl.kernel,
        mesh=mesh,
        out_type=jax.ShapeDtypeStruct((_B, _N, _D), jnp.float32),
        scratch_types=[
            pltpu.VMEM((_K, _G), jnp.int32),
            pltpu.VMEM((_G, _D), jnp.float32),
            pltpu.SemaphoreType.DMA,
        ],
    )
    def sc_kernel(y_hbm, idx_hbm, t0_hbm, out_hbm, idxv, accv, sem):
        cid = lax.axis_index("c")
        sid = lax.axis_index("s")
        w = sid * _NC + cid
        b = w // wpb
        nbase = (w % wpb) * rows_per_w

        def chunk(j, carry):
            n0 = nbase + j * _G
            pltpu.sync_copy(idx_hbm.at[b, :, pl.ds(n0, _G)], idxv)
            pltpu.sync_copy(t0_hbm.at[b, pl.ds(n0, _G)], accv)
            descs = [
                pltpu.async_copy(y_hbm.at[b, k].at[idxv.at[k]], accv, sem,
                                 add=True)
                for k in range(_K)
            ]
            for d in descs:
                d.wait()
            pltpu.sync_copy(accv, out_hbm.at[b, pl.ds(n0, _G)])
            return carry

        lax.fori_loop(0, chunks, chunk, 0)

    return sc_kernel(y, idx, t0)


def _stats_body(o_ref, s_ref):
    @pl.when(pl.program_id(0) == 0)
    def _():
        s_ref[...] = jnp.zeros_like(s_ref)

    blk = o_ref[...]
    s_ref[0:1, :] += jnp.sum(blk, axis=0, keepdims=True)
    s_ref[1:2, :] += jnp.sum(blk * blk, axis=0, keepdims=True)


def _stats_call(o2d):
    return pl.pallas_call(
        _stats_body,
        grid=(_B * _N // _SB,),
        in_specs=[pl.BlockSpec((_SB, _D), lambda i: (i, 0))],
        out_specs=pl.BlockSpec((8, _D), lambda i: (0, 0)),
        out_shape=jax.ShapeDtypeStruct((8, _D), jnp.float32),
    )(o2d)


def _norm_body(o_ref, s_ref, gb_ref, y_ref):
    inv_n = jnp.float32(1.0 / (_B * _N))
    mean = s_ref[0:1, :] * inv_n
    var = s_ref[1:2, :] * inv_n - mean * mean
    scale = gb_ref[0:1, :] * lax.rsqrt(var + 1e-5)
    bias = gb_ref[1:2, :] - scale * mean
    y_ref[0] = jnp.transpose(o_ref[0] * scale + bias, (1, 0))


def _norm_call(o3d, sums, gb):
    return pl.pallas_call(
        _norm_body,
        grid=(_B, _N // _SB),
        in_specs=[
            pl.BlockSpec((1, _SB, _D), lambda b, i: (b, i, 0)),
            pl.BlockSpec((8, _D), lambda b, i: (0, 0)),
            pl.BlockSpec((8, _D), lambda b, i: (0, 0)),
        ],
        out_specs=pl.BlockSpec((1, _D, _SB), lambda b, i: (b, 0, i)),
        out_shape=jax.ShapeDtypeStruct((_B, _D, _N), jnp.float32),
    )(o3d, sums, gb)


def kernel(x, pos, spectral_filter, bn_gamma, bn_beta):
    pos_pad = jnp.concatenate(
        [pos, jnp.zeros((_B, 8 - pos.shape[1], _N), jnp.float32)], axis=1)
    idx = jnp.transpose(_knn_call(pos_pad), (0, 2, 1))
    y, t0neg = _proj_call(x, spectral_filter)
    out = _sc_gather_call(y, idx, t0neg)
    sums = _stats_call(out.reshape(_B * _N, _D))
    gb = jnp.zeros((8, _D), jnp.float32)
    gb = gb.at[0].set(bn_gamma).at[1].set(bn_beta)
    return _norm_call(out, sums, gb)
